# Initial kernel scaffold; baseline (speedup 1.0000x reference)
import functools

import jax
import jax.numpy as jnp
from jax import lax
from jax.experimental import pallas as pl
from jax.experimental.pallas import tpu as pltpu
from jax.experimental.pallas import tpu_sc as plsc

N = 10000
E = 320000
D = 128

NC = 2
NS = 16
NW = NC * NS
EPW = E // NW
K = 80
CHUNKS = EPW // K
RPT = N // NS
ZR = 25
DW = 16

_mesh = plsc.VectorSubcoreMesh(
    core_axis_name="c", subcore_axis_name="s", num_cores=NC, num_subcores=NS)


def _zero_fill(zbuf, width):
    def row(i, _):
        for j in range(width // 16):
            zbuf[i, pl.ds(j * 16, 16)] = jnp.zeros((16,), jnp.float32)
        return 0
    lax.fori_loop(0, ZR, row, 0)


@functools.partial(
    pl.kernel,
    out_type=(jax.ShapeDtypeStruct((N, D), jnp.float32),
              jax.ShapeDtypeStruct((N, D), jnp.float32)),
    mesh=_mesh,
    scratch_types=[
        pltpu.VMEM_SHARED((N, D), jnp.float32),
        pltpu.VMEM((K,), jnp.int32),
        pltpu.VMEM((K,), jnp.int32),
        pltpu.VMEM((K, D), jnp.float32),
        pltpu.VMEM((ZR, D), jnp.float32),
        pltpu.SemaphoreType.DMA,
    ],
)
def _sc_agg(x_hbm, src_hbm, dst_hbm, acc0_hbm, acc1_hbm,
            acc_sh, sidx, didx, rows, zbuf, sem):
    cid = lax.axis_index("c")
    sid = lax.axis_index("s")
    wid = sid * NC + cid

    _zero_fill(zbuf, D)
    def za(i, _):
        pltpu.sync_copy(zbuf, acc_sh.at[pl.ds(sid * RPT + i * ZR, ZR)])
        return 0
    lax.fori_loop(0, RPT // ZR, za, 0)
    plsc.subcore_barrier()

    ebase = wid * EPW
    def chunk(i, _):
        base = ebase + i * K
        pltpu.sync_copy(src_hbm.at[pl.ds(base, K)], sidx)
        pltpu.async_copy(x_hbm.at[sidx], rows, sem).wait()
        pltpu.sync_copy(dst_hbm.at[pl.ds(base, K)], didx)
        pltpu.sync_copy(rows, acc_sh.at[didx], add=True)
        return 0
    lax.fori_loop(0, CHUNKS, chunk, 0)
    plsc.subcore_barrier()

    r0 = sid * RPT
    @pl.when(cid == 0)
    def _():
        pltpu.sync_copy(acc_sh.at[pl.ds(r0, RPT)], acc0_hbm.at[pl.ds(r0, RPT)])
    @pl.when(cid == 1)
    def _():
        pltpu.sync_copy(acc_sh.at[pl.ds(r0, RPT)], acc1_hbm.at[pl.ds(r0, RPT)])


@functools.partial(
    pl.kernel,
    out_type=(jax.ShapeDtypeStruct((N, DW), jnp.float32),
              jax.ShapeDtypeStruct((N, DW), jnp.float32)),
    mesh=_mesh,
    scratch_types=[
        pltpu.VMEM_SHARED((N, DW), jnp.float32),
        pltpu.VMEM((K,), jnp.int32),
        pltpu.VMEM((K, DW), jnp.float32),
        pltpu.VMEM((ZR, DW), jnp.float32),
    ],
)
def _sc_deg(dst_hbm, deg0_hbm, deg1_hbm, deg_sh, didx, ones_v, zbuf):
    cid = lax.axis_index("c")
    sid = lax.axis_index("s")
    wid = sid * NC + cid

    _zero_fill(zbuf, DW)
    def fill_ones(i, _):
        ones_v[i, pl.ds(0, 16)] = jnp.ones((16,), jnp.float32)
        return 0
    lax.fori_loop(0, K, fill_ones, 0)
    def zd(i, _):
        pltpu.sync_copy(zbuf, deg_sh.at[pl.ds(sid * RPT + i * ZR, ZR)])
        return 0
    lax.fori_loop(0, RPT // ZR, zd, 0)
    plsc.subcore_barrier()

    ebase = wid * EPW
    def chunk(i, _):
        base = ebase + i * K
        pltpu.sync_copy(dst_hbm.at[pl.ds(base, K)], didx)
        pltpu.sync_copy(ones_v, deg_sh.at[didx], add=True)
        return 0
    lax.fori_loop(0, CHUNKS, chunk, 0)
    plsc.subcore_barrier()

    r0 = sid * RPT
    @pl.when(cid == 0)
    def _():
        pltpu.sync_copy(deg_sh.at[pl.ds(r0, RPT)], deg0_hbm.at[pl.ds(r0, RPT)])
    @pl.when(cid == 1)
    def _():
        pltpu.sync_copy(deg_sh.at[pl.ds(r0, RPT)], deg1_hbm.at[pl.ds(r0, RPT)])


BN = 1000


def _dense_body(relu, h_ref, a0_ref, a1_ref, d0_ref, d1_ref,
                ws_ref, wn_ref, b_ref, o_ref):
    deg = jnp.maximum(d0_ref[:, 0:1] + d1_ref[:, 0:1], 1.0)
    hn = (a0_ref[...] + a1_ref[...]) / deg
    out = (
        jnp.dot(h_ref[...], ws_ref[...], preferred_element_type=jnp.float32,
                precision=lax.Precision.HIGHEST)
        + jnp.dot(hn, wn_ref[...], preferred_element_type=jnp.float32,
                  precision=lax.Precision.HIGHEST)
        + b_ref[...]
    )
    if relu:
        out = jnp.maximum(out, 0.0)
    o_ref[...] = out


def _dense(h, a0, a1, d0, d1, ws, wn, b, relu):
    return pl.pallas_call(
        functools.partial(_dense_body, relu),
        out_shape=jax.ShapeDtypeStruct((N, D), jnp.float32),
        grid=(N // BN,),
        in_specs=[
            pl.BlockSpec((BN, D), lambda i: (i, 0)),
            pl.BlockSpec((BN, D), lambda i: (i, 0)),
            pl.BlockSpec((BN, D), lambda i: (i, 0)),
            pl.BlockSpec((BN, DW), lambda i: (i, 0)),
            pl.BlockSpec((BN, DW), lambda i: (i, 0)),
            pl.BlockSpec((D, D), lambda i: (0, 0)),
            pl.BlockSpec((D, D), lambda i: (0, 0)),
            pl.BlockSpec((1, D), lambda i: (0, 0)),
        ],
        out_specs=pl.BlockSpec((BN, D), lambda i: (i, 0)),
    )(h, a0, a1, d0, d1, ws, wn, b)


def kernel(x, edge_index, W_self_1, W_neigh_1, b_1,
           W_self_2, W_neigh_2, b_2, W_self_3, W_neigh_3, b_3):
    src = edge_index[0].astype(jnp.int32)
    dst = edge_index[1].astype(jnp.int32)
    deg0, deg1 = _sc_deg(dst)
    h = x
    layers = [
        (W_self_1, W_neigh_1, b_1, True),
        (W_self_2, W_neigh_2, b_2, True),
        (W_self_3, W_neigh_3, b_3, False),
    ]
    for ws, wn, b, relu in layers:
        a0, a1 = _sc_agg(h, src, dst)
        h = _dense(h, a0, a1, deg0, deg1, ws, wn, b.reshape(1, D), relu)
    return h


# speedup vs baseline: 4.2975x; 4.2975x over previous
import functools

import jax
import jax.numpy as jnp
from jax import lax
from jax.experimental import pallas as pl
from jax.experimental.pallas import tpu as pltpu
from jax.experimental.pallas import tpu_sc as plsc

N = 10000
E = 320000
D = 128

NC = 2
NS = 16
NW = NC * NS
EPW = E // NW
K = 80
CHUNKS = EPW // K
RPT = 624
TAIL0 = NS * RPT
ZR = 16

_mesh = plsc.VectorSubcoreMesh(
    core_axis_name="c", subcore_axis_name="s", num_cores=NC, num_subcores=NS)


def _zero_fill(zbuf, width):
    def row(i, _):
        for j in range(width // 16):
            zbuf[i, pl.ds(j * 16, 16)] = jnp.zeros((16,), jnp.float32)
        return 0
    lax.fori_loop(0, ZR, row, 0)


def _zero_shared(sh, zbuf, sid):
    base = sid * RPT
    def z(i, _):
        pltpu.sync_copy(zbuf, sh.at[pl.ds(base + i * ZR, ZR)])
        return 0
    lax.fori_loop(0, 40, z, 0)


def _writeback(src_sh, dst_hbm, sid):
    r0 = sid * RPT
    pltpu.sync_copy(src_sh.at[pl.ds(r0, RPT)], dst_hbm.at[pl.ds(r0, RPT)])
    @pl.when(sid == NS - 1)
    def _():
        pltpu.sync_copy(src_sh.at[pl.ds(TAIL0, N - TAIL0)],
                        dst_hbm.at[pl.ds(TAIL0, N - TAIL0)])


@functools.partial(
    pl.kernel,
    out_type=(jax.ShapeDtypeStruct((N, D), jnp.float32),
              jax.ShapeDtypeStruct((N, D), jnp.float32)),
    mesh=_mesh,
    scratch_types=[
        pltpu.VMEM_SHARED((N, D), jnp.float32),
        pltpu.VMEM((K,), jnp.int32),
        pltpu.VMEM((K,), jnp.int32),
        pltpu.VMEM((K, D), jnp.float32),
        pltpu.VMEM((ZR, D), jnp.float32),
        pltpu.SemaphoreType.DMA,
    ],
)
def _sc_agg(x_hbm, src_hbm, dst_hbm, acc0_hbm, acc1_hbm,
            acc_sh, sidx, didx, rows, zbuf, sem):
    cid = lax.axis_index("c")
    sid = lax.axis_index("s")
    wid = sid * NC + cid

    _zero_fill(zbuf, D)
    _zero_shared(acc_sh, zbuf, sid)
    plsc.subcore_barrier()

    ebase = wid * EPW
    def chunk(i, _):
        base = ebase + i * K
        pltpu.sync_copy(src_hbm.at[pl.ds(base, K)], sidx)
        pltpu.async_copy(x_hbm.at[sidx], rows, sem).wait()
        pltpu.sync_copy(dst_hbm.at[pl.ds(base, K)], didx)
        pltpu.sync_copy(rows, acc_sh.at[didx], add=True)
        return 0
    lax.fori_loop(0, CHUNKS, chunk, 0)
    plsc.subcore_barrier()

    @pl.when(cid == 0)
    def _():
        _writeback(acc_sh, acc0_hbm, sid)
    @pl.when(cid == 1)
    def _():
        _writeback(acc_sh, acc1_hbm, sid)


@functools.partial(
    pl.kernel,
    out_type=(jax.ShapeDtypeStruct((N, D), jnp.float32),
              jax.ShapeDtypeStruct((N, D), jnp.float32)),
    mesh=_mesh,
    scratch_types=[
        pltpu.VMEM_SHARED((N, D), jnp.float32),
        pltpu.VMEM((K,), jnp.int32),
        pltpu.VMEM((K, D), jnp.float32),
        pltpu.VMEM((ZR, D), jnp.float32),
    ],
)
def _sc_deg(dst_hbm, deg0_hbm, deg1_hbm, deg_sh, didx, ones_v, zbuf):
    cid = lax.axis_index("c")
    sid = lax.axis_index("s")
    wid = sid * NC + cid

    _zero_fill(zbuf, D)
    def fill_ones(i, _):
        for j in range(D // 16):
            ones_v[i, pl.ds(j * 16, 16)] = jnp.ones((16,), jnp.float32)
        return 0
    lax.fori_loop(0, K, fill_ones, 0)
    _zero_shared(deg_sh, zbuf, sid)
    plsc.subcore_barrier()

    ebase = wid * EPW
    def chunk(i, _):
        base = ebase + i * K
        pltpu.sync_copy(dst_hbm.at[pl.ds(base, K)], didx)
        pltpu.sync_copy(ones_v, deg_sh.at[didx], add=True)
        return 0
    lax.fori_loop(0, CHUNKS, chunk, 0)
    plsc.subcore_barrier()

    @pl.when(cid == 0)
    def _():
        _writeback(deg_sh, deg0_hbm, sid)
    @pl.when(cid == 1)
    def _():
        _writeback(deg_sh, deg1_hbm, sid)


BN = 1000


def _dense_body(relu, h_ref, a0_ref, a1_ref, d0_ref, d1_ref,
                ws_ref, wn_ref, b_ref, o_ref):
    deg = jnp.maximum(d0_ref[:, 0:1] + d1_ref[:, 0:1], 1.0)
    hn = (a0_ref[...] + a1_ref[...]) / deg
    out = (
        jnp.dot(h_ref[...], ws_ref[...], preferred_element_type=jnp.float32,
                precision=lax.Precision.HIGHEST)
        + jnp.dot(hn, wn_ref[...], preferred_element_type=jnp.float32,
                  precision=lax.Precision.HIGHEST)
        + b_ref[...]
    )
    if relu:
        out = jnp.maximum(out, 0.0)
    o_ref[...] = out


def _dense(h, a0, a1, d0, d1, ws, wn, b, relu):
    return pl.pallas_call(
        functools.partial(_dense_body, relu),
        out_shape=jax.ShapeDtypeStruct((N, D), jnp.float32),
        grid=(N // BN,),
        in_specs=[
            pl.BlockSpec((BN, D), lambda i: (i, 0)),
            pl.BlockSpec((BN, D), lambda i: (i, 0)),
            pl.BlockSpec((BN, D), lambda i: (i, 0)),
            pl.BlockSpec((BN, D), lambda i: (i, 0)),
            pl.BlockSpec((BN, D), lambda i: (i, 0)),
            pl.BlockSpec((D, D), lambda i: (0, 0)),
            pl.BlockSpec((D, D), lambda i: (0, 0)),
            pl.BlockSpec((1, D), lambda i: (0, 0)),
        ],
        out_specs=pl.BlockSpec((BN, D), lambda i: (i, 0)),
    )(h, a0, a1, d0, d1, ws, wn, b)


def kernel(x, edge_index, W_self_1, W_neigh_1, b_1,
           W_self_2, W_neigh_2, b_2, W_self_3, W_neigh_3, b_3):
    src = edge_index[0].astype(jnp.int32)
    dst = edge_index[1].astype(jnp.int32)
    deg0, deg1 = _sc_deg(dst)
    h = x
    layers = [
        (W_self_1, W_neigh_1, b_1, True),
        (W_self_2, W_neigh_2, b_2, True),
        (W_self_3, W_neigh_3, b_3, False),
    ]
    for ws, wn, b, relu in layers:
        a0, a1 = _sc_agg(h, src, dst)
        h = _dense(h, a0, a1, deg0, deg1, ws, wn, b.reshape(1, D), relu)
    return h


# speedup vs baseline: 7.7531x; 1.8041x over previous
import functools

import jax
import jax.numpy as jnp
from jax import lax
from jax.experimental import pallas as pl
from jax.experimental.pallas import tpu as pltpu
from jax.experimental.pallas import tpu_sc as plsc

N = 10000
E = 320000
D = 128

NC = 2
NS = 16
NW = NC * NS
EPW = E // NW
K = 128
CHUNKS = EPW // K
TK = EPW - CHUNKS * K
PAIRS = CHUNKS // 2
RPT = 624
TAIL0 = NS * RPT
ZR = 80

_mesh = plsc.VectorSubcoreMesh(
    core_axis_name="c", subcore_axis_name="s", num_cores=NC, num_subcores=NS)


def _zero_fill(zbuf, width):
    def row(i, _):
        for j in range(width // 16):
            zbuf[i, pl.ds(j * 16, 16)] = jnp.zeros((16,), jnp.float32)
        return 0
    lax.fori_loop(0, ZR, row, 0)


def _zero_shared(sh, zbuf, sid):
    base = sid * RPT
    def z(i, _):
        pltpu.sync_copy(zbuf, sh.at[pl.ds(base + i * ZR, ZR)])
        return 0
    lax.fori_loop(0, 640 // ZR, z, 0)


def _writeback(src_sh, dst_hbm, sid):
    r0 = sid * RPT
    pltpu.sync_copy(src_sh.at[pl.ds(r0, RPT)], dst_hbm.at[pl.ds(r0, RPT)])
    @pl.when(sid == NS - 1)
    def _():
        pltpu.sync_copy(src_sh.at[pl.ds(TAIL0, N - TAIL0)],
                        dst_hbm.at[pl.ds(TAIL0, N - TAIL0)])


@functools.partial(
    pl.kernel,
    out_type=(jax.ShapeDtypeStruct((N, D), jnp.float32),
              jax.ShapeDtypeStruct((N, D), jnp.float32)),
    mesh=_mesh,
    scratch_types=[
        pltpu.VMEM_SHARED((N, D), jnp.float32),
        pltpu.VMEM((K,), jnp.int32),
        pltpu.VMEM((K,), jnp.int32),
        pltpu.VMEM((K,), jnp.int32),
        pltpu.VMEM((K,), jnp.int32),
        pltpu.VMEM((TK,), jnp.int32),
        pltpu.VMEM((TK,), jnp.int32),
        pltpu.VMEM((K, D), jnp.float32),
        pltpu.VMEM((K, D), jnp.float32),
        pltpu.VMEM((TK, D), jnp.float32),
        pltpu.VMEM((ZR, D), jnp.float32),
        pltpu.SemaphoreType.DMA,
        pltpu.SemaphoreType.DMA,
        pltpu.SemaphoreType.DMA,
        pltpu.SemaphoreType.DMA,
    ],
)
def _sc_agg(x_hbm, src_hbm, dst_hbm, acc0_hbm, acc1_hbm,
            acc_sh, sidx0, sidx1, didx0, didx1, sidxt, didxt,
            rows0, rows1, rowst, zbuf, semg0, semg1, sems0, sems1):
    cid = lax.axis_index("c")
    sid = lax.axis_index("s")
    wid = sid * NC + cid

    _zero_fill(zbuf, D)
    _zero_shared(acc_sh, zbuf, sid)
    plsc.subcore_barrier()

    ebase = wid * EPW

    def fire_gather(c, sidx, rows, semg):
        pltpu.sync_copy(src_hbm.at[pl.ds(ebase + c * K, K)], sidx)
        pltpu.async_copy(x_hbm.at[sidx], rows, semg)

    fire_gather(0, sidx0, rows0, semg0)
    fire_gather(1, sidx1, rows1, semg1)

    def pair(p, _):
        c = 2 * p
        pltpu.make_async_copy(x_hbm.at[sidx0], rows0, semg0).wait()
        pltpu.sync_copy(dst_hbm.at[pl.ds(ebase + c * K, K)], didx0)
        s0 = pltpu.async_copy(rows0, acc_sh.at[didx0], sems0, add=True)
        pltpu.make_async_copy(x_hbm.at[sidx1], rows1, semg1).wait()
        pltpu.sync_copy(dst_hbm.at[pl.ds(ebase + (c + 1) * K, K)], didx1)
        s1 = pltpu.async_copy(rows1, acc_sh.at[didx1], sems1, add=True)
        s0.wait()
        @pl.when(c + 2 < CHUNKS)
        def _():
            fire_gather(c + 2, sidx0, rows0, semg0)
        s1.wait()
        @pl.when(c + 3 < CHUNKS)
        def _():
            fire_gather(c + 3, sidx1, rows1, semg1)
        return 0
    lax.fori_loop(0, PAIRS, pair, 0)

    tbase = ebase + CHUNKS * K
    pltpu.sync_copy(src_hbm.at[pl.ds(tbase, TK)], sidxt)
    pltpu.async_copy(x_hbm.at[sidxt], rowst, semg0).wait()
    pltpu.sync_copy(dst_hbm.at[pl.ds(tbase, TK)], didxt)
    pltpu.sync_copy(rowst, acc_sh.at[didxt], add=True)
    plsc.subcore_barrier()

    @pl.when(cid == 0)
    def _():
        _writeback(acc_sh, acc0_hbm, sid)
    @pl.when(cid == 1)
    def _():
        _writeback(acc_sh, acc1_hbm, sid)


@functools.partial(
    pl.kernel,
    out_type=(jax.ShapeDtypeStruct((N, D), jnp.float32),
              jax.ShapeDtypeStruct((N, D), jnp.float32)),
    mesh=_mesh,
    scratch_types=[
        pltpu.VMEM_SHARED((N, D), jnp.float32),
        pltpu.VMEM((K,), jnp.int32),
        pltpu.VMEM((TK,), jnp.int32),
        pltpu.VMEM((K, D), jnp.float32),
        pltpu.VMEM((ZR, D), jnp.float32),
    ],
)
def _sc_deg(dst_hbm, deg0_hbm, deg1_hbm, deg_sh, didx, didxt, ones_v, zbuf):
    cid = lax.axis_index("c")
    sid = lax.axis_index("s")
    wid = sid * NC + cid

    _zero_fill(zbuf, D)
    def fill_ones(i, _):
        for j in range(D // 16):
            ones_v[i, pl.ds(j * 16, 16)] = jnp.ones((16,), jnp.float32)
        return 0
    lax.fori_loop(0, K, fill_ones, 0)
    _zero_shared(deg_sh, zbuf, sid)
    plsc.subcore_barrier()

    ebase = wid * EPW
    def chunk(i, _):
        base = ebase + i * K
        pltpu.sync_copy(dst_hbm.at[pl.ds(base, K)], didx)
        pltpu.sync_copy(ones_v, deg_sh.at[didx], add=True)
        return 0
    lax.fori_loop(0, CHUNKS, chunk, 0)
    tbase = ebase + CHUNKS * K
    pltpu.sync_copy(dst_hbm.at[pl.ds(tbase, TK)], didxt)
    pltpu.sync_copy(ones_v.at[pl.ds(0, TK)], deg_sh.at[didxt], add=True)
    plsc.subcore_barrier()

    @pl.when(cid == 0)
    def _():
        _writeback(deg_sh, deg0_hbm, sid)
    @pl.when(cid == 1)
    def _():
        _writeback(deg_sh, deg1_hbm, sid)


BN = 1000


def _dense_body(relu, h_ref, a0_ref, a1_ref, d0_ref, d1_ref,
                ws_ref, wn_ref, b_ref, o_ref):
    deg = jnp.maximum(d0_ref[:, 0:1] + d1_ref[:, 0:1], 1.0)
    hn = (a0_ref[...] + a1_ref[...]) / deg
    out = (
        jnp.dot(h_ref[...], ws_ref[...], preferred_element_type=jnp.float32,
                precision=lax.Precision.HIGHEST)
        + jnp.dot(hn, wn_ref[...], preferred_element_type=jnp.float32,
                  precision=lax.Precision.HIGHEST)
        + b_ref[...]
    )
    if relu:
        out = jnp.maximum(out, 0.0)
    o_ref[...] = out


def _dense(h, a0, a1, d0, d1, ws, wn, b, relu):
    return pl.pallas_call(
        functools.partial(_dense_body, relu),
        out_shape=jax.ShapeDtypeStruct((N, D), jnp.float32),
        grid=(N // BN,),
        in_specs=[
            pl.BlockSpec((BN, D), lambda i: (i, 0)),
            pl.BlockSpec((BN, D), lambda i: (i, 0)),
            pl.BlockSpec((BN, D), lambda i: (i, 0)),
            pl.BlockSpec((BN, D), lambda i: (i, 0)),
            pl.BlockSpec((BN, D), lambda i: (i, 0)),
            pl.BlockSpec((D, D), lambda i: (0, 0)),
            pl.BlockSpec((D, D), lambda i: (0, 0)),
            pl.BlockSpec((1, D), lambda i: (0, 0)),
        ],
        out_specs=pl.BlockSpec((BN, D), lambda i: (i, 0)),
    )(h, a0, a1, d0, d1, ws, wn, b)


def kernel(x, edge_index, W_self_1, W_neigh_1, b_1,
           W_self_2, W_neigh_2, b_2, W_self_3, W_neigh_3, b_3):
    src = edge_index[0].astype(jnp.int32)
    dst = edge_index[1].astype(jnp.int32)
    deg0, deg1 = _sc_deg(dst)
    h = x
    layers = [
        (W_self_1, W_neigh_1, b_1, True),
        (W_self_2, W_neigh_2, b_2, True),
        (W_self_3, W_neigh_3, b_3, False),
    ]
    for ws, wn, b, relu in layers:
        a0, a1 = _sc_agg(h, src, dst)
        h = _dense(h, a0, a1, deg0, deg1, ws, wn, b.reshape(1, D), relu)
    return h
